# baseline (device time: 62941 ns/iter reference)
import jax
import jax.numpy as jnp
from jax import lax
from jax.experimental import pallas as pl
from jax.experimental.pallas import tpu as pltpu

N_DEV = 8
N_HOPS = N_DEV - 1
N_SUB = 2


def _ring(q):
    return jnp.where(q < 4, q, 11 - q)


def kernel(x, dy):
    m, d = x.shape
    _, f = dy.shape
    ch = d // N_DEV
    hf = f // 2
    qf = hf // N_SUB

    xt = x.astype(jnp.bfloat16).T
    dy_bf = dy.astype(jnp.bfloat16)

    def body(xt_ref, dy_ref, out_ref, p_ref,
             send_cw, recv_cw, send_ccw, recv_ccw,
             ss_cw, rs_cw, ss_ccw, rs_ccw):
        j = lax.axis_index("i")
        r = _ring(j)
        succ = _ring((r + 1) % N_DEV)
        pred = _ring((r - 1) % N_DEV)

        barrier = pltpu.get_barrier_semaphore()
        for nbr in (succ, pred):
            pl.semaphore_signal(barrier, inc=1, device_id=(nbr,),
                                device_id_type=pl.DeviceIdType.MESH)
        pl.semaphore_wait(barrier, 2)

        def partial_half(q, lo):
            return lax.dot_general(
                xt_ref[pl.ds(_ring(q) * ch, ch), :],
                dy_ref[:, pl.ds(lo, hf)],
                (((1,), (0,)), ((), ())),
                preferred_element_type=jnp.float32,
            ).astype(jnp.bfloat16)

        streams = []
        for b in range(N_SUB):
            streams.append(dict(
                t=b, send=send_cw, recv=recv_cw, ss=ss_cw, rs=rs_cw,
                tgt=succ, b=b, col=b * qf,
                q_arr=lambda s: (r - 2 - s) % N_DEV,
            ))
        for b in range(N_SUB):
            streams.append(dict(
                t=N_SUB + b, send=send_ccw, recv=recv_ccw, ss=ss_ccw,
                rs=rs_ccw, tgt=pred, b=b, col=hf + b * qf,
                q_arr=lambda s: (r + 2 + s) % N_DEV,
            ))

        def make_rdma(st, s):
            slot = s % 2
            return pltpu.make_async_remote_copy(
                src_ref=st["send"].at[slot, st["b"]],
                dst_ref=st["recv"].at[s, st["b"]],
                send_sem=st["ss"].at[slot, st["b"]],
                recv_sem=st["rs"].at[s, st["b"]],
                device_id=(st["tgt"],),
                device_id_type=pl.DeviceIdType.MESH,
            )

        h_cw = partial_half((r - 1) % N_DEV, 0)
        send_cw[0, 0, :, :] = h_cw[:, :qf]
        send_cw[0, 1, :, :] = h_cw[:, qf:]
        h_ccw = partial_half((r + 1) % N_DEV, hf)
        send_ccw[0, 0, :, :] = h_ccw[:, :qf]
        send_ccw[0, 1, :, :] = h_ccw[:, qf:]

        inflight = {}
        for st in streams:
            rdma = make_rdma(st, 0)
            rdma.start()
            inflight[(st["t"], 0)] = rdma

        p_ref[:, :] = lax.dot_general(
            xt_ref[:, :], dy_ref[:, :],
            (((1,), (0,)), ((), ())),
            preferred_element_type=jnp.float32,
        ).astype(jnp.bfloat16)

        for s in range(N_HOPS):
            for st in streams:
                t, b = st["t"], st["b"]
                local = p_ref[pl.ds(_ring(st["q_arr"](s)) * ch, ch),
                              pl.ds(st["col"], qf)]
                inflight[(t, s)].wait_recv()
                acc = (local.astype(jnp.float32)
                       + st["recv"][s, b, :, :].astype(jnp.float32))
                if s < N_HOPS - 1:
                    if s >= 1:
                        inflight[(t, s - 1)].wait_send()
                    st["send"][(s + 1) % 2, b, :, :] = acc.astype(jnp.bfloat16)
                    rdma = make_rdma(st, s + 1)
                    rdma.start()
                    inflight[(t, s + 1)] = rdma
                else:
                    out_ref[:, pl.ds(st["col"], qf)] = acc

        for st in streams:
            inflight[(st["t"], N_HOPS - 2)].wait_send()
            inflight[(st["t"], N_HOPS - 1)].wait_send()

    return pl.pallas_call(
        body,
        out_shape=jax.ShapeDtypeStruct((ch, f), jnp.float32),
        in_specs=[
            pl.BlockSpec(memory_space=pltpu.VMEM),
            pl.BlockSpec(memory_space=pltpu.VMEM),
        ],
        out_specs=pl.BlockSpec(memory_space=pltpu.VMEM),
        scratch_shapes=[
            pltpu.VMEM((d, f), jnp.bfloat16),
            pltpu.VMEM((2, N_SUB, ch, qf), jnp.bfloat16),
            pltpu.VMEM((N_HOPS, N_SUB, ch, qf), jnp.bfloat16),
            pltpu.VMEM((2, N_SUB, ch, qf), jnp.bfloat16),
            pltpu.VMEM((N_HOPS, N_SUB, ch, qf), jnp.bfloat16),
            pltpu.SemaphoreType.DMA((2, N_SUB)),
            pltpu.SemaphoreType.DMA((N_HOPS, N_SUB)),
            pltpu.SemaphoreType.DMA((2, N_SUB)),
            pltpu.SemaphoreType.DMA((N_HOPS, N_SUB)),
        ],
        compiler_params=pltpu.CompilerParams(collective_id=0),
    )(xt, dy_bf)


# device time: 57102 ns/iter; 1.1023x vs baseline; 1.1023x over previous
import jax
import jax.numpy as jnp
from jax import lax
from jax.experimental import pallas as pl
from jax.experimental.pallas import tpu as pltpu

N_DEV = 8
N_HOPS = N_DEV - 1
N_SUB = 4


def _ring(q):
    return jnp.where(q < 4, q, 11 - q)


def kernel(x, dy):
    m, d = x.shape
    _, f = dy.shape
    ch = d // N_DEV
    hf = f // 2
    qf = hf // N_SUB

    xt = x.astype(jnp.bfloat16).T
    dy_bf = dy.astype(jnp.bfloat16)

    def body(xt_ref, dy_ref, out_ref, tmp,
             send_cw, recv_cw, send_ccw, recv_ccw,
             ss_cw, rs_cw, ss_ccw, rs_ccw):
        j = lax.axis_index("i")
        r = _ring(j)
        succ = _ring((r + 1) % N_DEV)
        pred = _ring((r - 1) % N_DEV)

        barrier = pltpu.get_barrier_semaphore()
        for nbr in (succ, pred):
            pl.semaphore_signal(barrier, inc=1, device_id=(nbr,),
                                device_id_type=pl.DeviceIdType.MESH)
        pl.semaphore_wait(barrier, 2)

        def partial(q, lo):
            return lax.dot_general(
                xt_ref[pl.ds(_ring(q) * ch, ch), :],
                dy_ref[:, pl.ds(lo, qf)],
                (((1,), (0,)), ((), ())),
                preferred_element_type=jnp.float32,
            )

        streams = []
        for b in range(N_SUB):
            streams.append(dict(
                t=b, send=send_cw, recv=recv_cw, ss=ss_cw, rs=rs_cw,
                tgt=succ, b=b, col=b * qf,
                q_send=lambda s: (r - 1 - s) % N_DEV,
                q_arr=lambda s: (r - 2 - s) % N_DEV,
            ))
            streams.append(dict(
                t=N_SUB + b, send=send_ccw, recv=recv_ccw, ss=ss_ccw,
                rs=rs_ccw, tgt=pred, b=b, col=hf + b * qf,
                q_send=lambda s: (r + 1 + s) % N_DEV,
                q_arr=lambda s: (r + 2 + s) % N_DEV,
            ))

        def make_rdma(st, s):
            slot = s % 2
            return pltpu.make_async_remote_copy(
                src_ref=st["send"].at[slot, st["b"]],
                dst_ref=st["recv"].at[s, st["b"]],
                send_sem=st["ss"].at[slot, st["b"]],
                recv_sem=st["rs"].at[s, st["b"]],
                device_id=(st["tgt"],),
                device_id_type=pl.DeviceIdType.MESH,
            )

        inflight = {}
        for st in streams:
            st["send"][0, st["b"], :, :] = partial(
                st["q_send"](0), st["col"]).astype(jnp.bfloat16)
            rdma = make_rdma(st, 0)
            rdma.start()
            inflight[(st["t"], 0)] = rdma
        for st in streams:
            tmp[st["t"], :, :] = partial(st["q_arr"](0), st["col"])

        for s in range(N_HOPS):
            for st in streams:
                t, b = st["t"], st["b"]
                inflight[(t, s)].wait_recv()
                acc = tmp[t, :, :] + st["recv"][s, b, :, :].astype(jnp.float32)
                if s < N_HOPS - 1:
                    if s >= 1:
                        inflight[(t, s - 1)].wait_send()
                    st["send"][(s + 1) % 2, b, :, :] = acc.astype(jnp.bfloat16)
                    rdma = make_rdma(st, s + 1)
                    rdma.start()
                    inflight[(t, s + 1)] = rdma
                    tmp[t, :, :] = partial(st["q_arr"](s + 1), st["col"])
                else:
                    out_ref[:, pl.ds(st["col"], qf)] = acc

        for st in streams:
            inflight[(st["t"], N_HOPS - 2)].wait_send()
            inflight[(st["t"], N_HOPS - 1)].wait_send()

    return pl.pallas_call(
        body,
        out_shape=jax.ShapeDtypeStruct((ch, f), jnp.float32),
        in_specs=[
            pl.BlockSpec(memory_space=pltpu.VMEM),
            pl.BlockSpec(memory_space=pltpu.VMEM),
        ],
        out_specs=pl.BlockSpec(memory_space=pltpu.VMEM),
        scratch_shapes=[
            pltpu.VMEM((2 * N_SUB, ch, qf), jnp.float32),
            pltpu.VMEM((2, N_SUB, ch, qf), jnp.bfloat16),
            pltpu.VMEM((N_HOPS, N_SUB, ch, qf), jnp.bfloat16),
            pltpu.VMEM((2, N_SUB, ch, qf), jnp.bfloat16),
            pltpu.VMEM((N_HOPS, N_SUB, ch, qf), jnp.bfloat16),
            pltpu.SemaphoreType.DMA((2, N_SUB)),
            pltpu.SemaphoreType.DMA((N_HOPS, N_SUB)),
            pltpu.SemaphoreType.DMA((2, N_SUB)),
            pltpu.SemaphoreType.DMA((N_HOPS, N_SUB)),
        ],
        compiler_params=pltpu.CompilerParams(collective_id=0),
    )(xt, dy_bf)


# device time: 56145 ns/iter; 1.1210x vs baseline; 1.0170x over previous
import jax
import jax.numpy as jnp
from jax import lax
from jax.experimental import pallas as pl
from jax.experimental.pallas import tpu as pltpu

N_DEV = 8
N_HOPS = N_DEV - 1
N_SUB = 4


def _ring(q):
    return jnp.where(q < 4, q, 11 - q)


def kernel(x, dy):
    m, d = x.shape
    _, f = dy.shape
    ch = d // N_DEV
    hf = f // 2
    qf = hf // N_SUB

    x_bf = x.astype(jnp.bfloat16)
    dy_bf = dy.astype(jnp.bfloat16)

    def body(x_ref, dy_ref, out_ref, tmp,
             send_cw, recv_cw, send_ccw, recv_ccw,
             ss_cw, rs_cw, ss_ccw, rs_ccw):
        j = lax.axis_index("i")
        r = _ring(j)
        succ = _ring((r + 1) % N_DEV)
        pred = _ring((r - 1) % N_DEV)

        barrier = pltpu.get_barrier_semaphore()
        for nbr in (succ, pred):
            pl.semaphore_signal(barrier, inc=1, device_id=(nbr,),
                                device_id_type=pl.DeviceIdType.MESH)
        pl.semaphore_wait(barrier, 2)

        def partial(q, lo):
            return lax.dot_general(
                x_ref[:, pl.ds(_ring(q) * ch, ch)],
                dy_ref[:, pl.ds(lo, qf)],
                (((0,), (0,)), ((), ())),
                preferred_element_type=jnp.float32,
            )

        streams = []
        for b in range(N_SUB):
            streams.append(dict(
                t=b, send=send_cw, recv=recv_cw, ss=ss_cw, rs=rs_cw,
                tgt=succ, b=b, col=b * qf,
                q_send=lambda s: (r - 1 - s) % N_DEV,
                q_arr=lambda s: (r - 2 - s) % N_DEV,
            ))
            streams.append(dict(
                t=N_SUB + b, send=send_ccw, recv=recv_ccw, ss=ss_ccw,
                rs=rs_ccw, tgt=pred, b=b, col=hf + b * qf,
                q_send=lambda s: (r + 1 + s) % N_DEV,
                q_arr=lambda s: (r + 2 + s) % N_DEV,
            ))

        def make_rdma(st, s):
            slot = s % 2
            return pltpu.make_async_remote_copy(
                src_ref=st["send"].at[slot, st["b"]],
                dst_ref=st["recv"].at[s, st["b"]],
                send_sem=st["ss"].at[slot, st["b"]],
                recv_sem=st["rs"].at[s, st["b"]],
                device_id=(st["tgt"],),
                device_id_type=pl.DeviceIdType.MESH,
            )

        inflight = {}
        for st in streams:
            st["send"][0, st["b"], :, :] = partial(
                st["q_send"](0), st["col"]).astype(jnp.bfloat16)
            rdma = make_rdma(st, 0)
            rdma.start()
            inflight[(st["t"], 0)] = rdma
        for st in streams:
            tmp[st["t"], :, :] = partial(st["q_arr"](0), st["col"])

        for s in range(N_HOPS):
            for st in streams:
                t, b = st["t"], st["b"]
                inflight[(t, s)].wait_recv()
                acc = tmp[t, :, :] + st["recv"][s, b, :, :].astype(jnp.float32)
                if s < N_HOPS - 1:
                    if s >= 1:
                        inflight[(t, s - 1)].wait_send()
                    st["send"][(s + 1) % 2, b, :, :] = acc.astype(jnp.bfloat16)
                    rdma = make_rdma(st, s + 1)
                    rdma.start()
                    inflight[(t, s + 1)] = rdma
                    tmp[t, :, :] = partial(st["q_arr"](s + 1), st["col"])
                else:
                    out_ref[:, pl.ds(st["col"], qf)] = acc

        for st in streams:
            inflight[(st["t"], N_HOPS - 2)].wait_send()
            inflight[(st["t"], N_HOPS - 1)].wait_send()

    return pl.pallas_call(
        body,
        out_shape=jax.ShapeDtypeStruct((ch, f), jnp.float32),
        in_specs=[
            pl.BlockSpec(memory_space=pltpu.VMEM),
            pl.BlockSpec(memory_space=pltpu.VMEM),
        ],
        out_specs=pl.BlockSpec(memory_space=pltpu.VMEM),
        scratch_shapes=[
            pltpu.VMEM((2 * N_SUB, ch, qf), jnp.float32),
            pltpu.VMEM((2, N_SUB, ch, qf), jnp.bfloat16),
            pltpu.VMEM((N_HOPS, N_SUB, ch, qf), jnp.bfloat16),
            pltpu.VMEM((2, N_SUB, ch, qf), jnp.bfloat16),
            pltpu.VMEM((N_HOPS, N_SUB, ch, qf), jnp.bfloat16),
            pltpu.SemaphoreType.DMA((2, N_SUB)),
            pltpu.SemaphoreType.DMA((N_HOPS, N_SUB)),
            pltpu.SemaphoreType.DMA((2, N_SUB)),
            pltpu.SemaphoreType.DMA((N_HOPS, N_SUB)),
        ],
        compiler_params=pltpu.CompilerParams(collective_id=0),
    )(x_bf, dy_bf)
